# baseline (speedup 1.0000x reference)
import jax
import jax.numpy as jnp
from jax import lax
from jax.experimental import pallas as pl
from jax.experimental.pallas import tpu as pltpu
from jax.experimental.pallas import tpu_sc as plsc

_NUM_NODES = 50000
_EMBED_DIM = 64
_NUM_EDGES = 800000

_B = 2 * _NUM_EDGES
_NC = 2
_NS = 16
_NW = _NC * _NS
_C = 128
_RB = _EMBED_DIM // 8
_TILES_PER_I = _NUM_EDGES // _C
_TPW = _TILES_PER_I // _NS
_EXTRA = _TILES_PER_I - _TPW * _NS


def _body(idx_hbm, table_hbm, out_hbm, idx_v, rows_v, tr_v, gsem, wsem):
    wid = lax.axis_index("s") * _NC + lax.axis_index("c")
    i = wid // _NS
    wp = wid % _NS
    s_tile = wp * _TPW + lax.min(wp, _EXTRA)
    cnt = _TPW + jnp.where(wp < _EXTRA, 1, 0)
    flat_base = i * _NUM_EDGES + s_tile * _C

    pltpu.sync_copy(idx_hbm.at[pl.ds(flat_base, _TPW * _C)],
                    idx_v.at[pl.ds(0, _TPW * _C)])

    @pl.when(cnt > _TPW)
    def _():
        pltpu.sync_copy(idx_hbm.at[pl.ds(flat_base + _TPW * _C, _C)],
                        idx_v.at[pl.ds(_TPW * _C, _C)])

    def fire_gather(j, slot):
        pltpu.async_copy(
            table_hbm.at[idx_v.at[pl.ds(j * _C, _C)]],
            rows_v.at[pl.ds(slot * _C, _C)],
            gsem.at[slot],
        )

    def drain_gather(slot):
        pltpu.make_async_copy(
            table_hbm.at[pl.ds(0, _C)],
            rows_v.at[pl.ds(slot * _C, _C)],
            gsem.at[slot],
        ).wait()

    def fire_writeback(j, half):
        for rb in range(_RB):
            pltpu.async_copy(
                tr_v.at[half * _RB + rb],
                out_hbm.at[i, rb, s_tile + j],
                wsem.at[half],
            )

    def drain_writeback(half):
        pltpu.make_async_copy(
            table_hbm.at[pl.ds(0, _C)],
            rows_v.at[pl.ds(half * _C, _C)],
            wsem.at[half],
        ).wait()

    fire_gather(0, 0)

    @pl.when(cnt >= 2)
    def _():
        fire_gather(1, 1)

    @pl.when(cnt >= 3)
    def _():
        fire_gather(2, 2)

    def step(j, carry):
        h = lax.rem(j, 2)
        g3 = lax.rem(j, 4)

        @pl.when(j + 3 < cnt)
        def _():
            fire_gather(j + 3, lax.rem(j + 3, 4))

        drain_gather(g3)

        @pl.when(j >= 2)
        def _():
            drain_writeback(h)

        row_off = g3 * _C
        diota = lax.iota(jnp.int32, 16)
        e_locs = [diota + (g * 16) for g in range(_C // 16)]
        e_idxs = [e_loc + row_off for e_loc in e_locs]
        h8 = h * _RB
        for d0 in range(_EMBED_DIM):
            col = (diota + d0) & 63
            dhi = (col >> 3) + h8
            dlo = col & 7
            vs = [
                plsc.load_gather(rows_v, [e_idx, col]) for e_idx in e_idxs
            ]
            for e_loc, v in zip(e_locs, vs):
                plsc.store_scatter(tr_v, [dhi, dlo, e_loc], v)

        fire_writeback(j, h)
        return carry

    lax.fori_loop(0, cnt, step, 0)

    drain_writeback(lax.rem(cnt, 2))

    @pl.when(cnt >= 2)
    def _():
        drain_writeback(lax.rem(cnt + 1, 2))


@jax.jit
def _gather(idx, table):
    mesh = plsc.VectorSubcoreMesh(core_axis_name="c", subcore_axis_name="s")
    return pl.kernel(
        _body,
        out_type=jax.ShapeDtypeStruct(
            (2, _RB, _TILES_PER_I, 8, _C), jnp.float32),
        mesh=mesh,
        scratch_types=[
            pltpu.VMEM(((_TPW + 1) * _C,), jnp.int32),
            pltpu.VMEM((4 * _C, _EMBED_DIM), jnp.float32),
            pltpu.VMEM((2 * _RB, 8, _C), jnp.float32),
            pltpu.SemaphoreType.DMA((4,)),
            pltpu.SemaphoreType.DMA((2,)),
        ],
        compiler_params=pltpu.CompilerParams(
            use_tc_tiling_on_sc=False, needs_layout_passes=False),
    )(idx, table)


def kernel(edge_index, embedding_weight):
    idx = edge_index.reshape(-1).astype(jnp.int32)
    out5 = _gather(idx, embedding_weight)
    return jnp.transpose(out5, (0, 2, 4, 1, 3)).reshape(2, _NUM_EDGES, _EMBED_DIM)


# speedup vs baseline: 1.0063x; 1.0063x over previous
import jax
import jax.numpy as jnp
from jax import lax
from jax.experimental import pallas as pl
from jax.experimental.pallas import tpu as pltpu
from jax.experimental.pallas import tpu_sc as plsc

_NUM_NODES = 50000
_EMBED_DIM = 64
_NUM_EDGES = 800000

_B = 2 * _NUM_EDGES
_NC = 2
_NS = 16
_NW = _NC * _NS
_C = 128
_RB = _EMBED_DIM // 8
_TILES_PER_I = _NUM_EDGES // _C
_TPW = _TILES_PER_I // _NS
_EXTRA = _TILES_PER_I - _TPW * _NS


def _body(idx_hbm, table_hbm, out_hbm, idx_v, rows_v, tr_v, gsem, wsem):
    wid = lax.axis_index("s") * _NC + lax.axis_index("c")
    i = wid // _NS
    wp = wid % _NS
    s_tile = wp * _TPW + lax.min(wp, _EXTRA)
    cnt = _TPW + jnp.where(wp < _EXTRA, 1, 0)
    flat_base = i * _NUM_EDGES + s_tile * _C

    pltpu.sync_copy(idx_hbm.at[pl.ds(flat_base, _TPW * _C)],
                    idx_v.at[pl.ds(0, _TPW * _C)])

    @pl.when(cnt > _TPW)
    def _():
        pltpu.sync_copy(idx_hbm.at[pl.ds(flat_base + _TPW * _C, _C)],
                        idx_v.at[pl.ds(_TPW * _C, _C)])

    def fire_gather(j, slot):
        pltpu.async_copy(
            table_hbm.at[idx_v.at[pl.ds(j * _C, _C)]],
            rows_v.at[pl.ds(slot * _C, _C)],
            gsem.at[slot],
        )

    def drain_gather(slot):
        pltpu.make_async_copy(
            table_hbm.at[pl.ds(0, _C)],
            rows_v.at[pl.ds(slot * _C, _C)],
            gsem.at[slot],
        ).wait()

    def fire_writeback(j, half):
        for rb in range(_RB):
            pltpu.async_copy(
                tr_v.at[half * _RB + rb],
                out_hbm.at[i, rb, s_tile + j],
                wsem.at[half],
            )

    def drain_writeback(half):
        pltpu.make_async_copy(
            table_hbm.at[pl.ds(0, _C)],
            rows_v.at[pl.ds(half * _C, _C)],
            wsem.at[half],
        ).wait()

    fire_gather(0, 0)

    @pl.when(cnt >= 2)
    def _():
        fire_gather(1, 1)

    def step(j, carry):
        h = lax.rem(j, 2)
        g3 = lax.rem(j, 3)

        @pl.when(j + 2 < cnt)
        def _():
            fire_gather(j + 2, lax.rem(j + 2, 3))

        drain_gather(g3)

        @pl.when(j >= 2)
        def _():
            drain_writeback(h)

        row_off = g3 * _C
        diota = lax.iota(jnp.int32, 16)
        e_locs = [diota + (g * 16) for g in range(_C // 16)]
        e_idxs = [e_loc + row_off for e_loc in e_locs]
        h8 = h * _RB
        for d0 in range(_EMBED_DIM):
            col = (diota + d0) & 63
            dhi = (col >> 3) + h8
            dlo = col & 7
            vs = [
                plsc.load_gather(rows_v, [e_idx, col]) for e_idx in e_idxs
            ]
            for e_loc, v in zip(e_locs, vs):
                plsc.store_scatter(tr_v, [dhi, dlo, e_loc], v)

        fire_writeback(j, h)
        return carry

    lax.fori_loop(0, cnt, step, 0)

    drain_writeback(lax.rem(cnt, 2))

    @pl.when(cnt >= 2)
    def _():
        drain_writeback(lax.rem(cnt + 1, 2))


@jax.jit
def _gather(idx, table):
    mesh = plsc.VectorSubcoreMesh(core_axis_name="c", subcore_axis_name="s")
    return pl.kernel(
        _body,
        out_type=jax.ShapeDtypeStruct(
            (2, _RB, _TILES_PER_I, 8, _C), jnp.float32),
        mesh=mesh,
        scratch_types=[
            pltpu.VMEM(((_TPW + 1) * _C,), jnp.int32),
            pltpu.VMEM((3 * _C, _EMBED_DIM), jnp.float32),
            pltpu.VMEM((2 * _RB, 8, _C), jnp.float32),
            pltpu.SemaphoreType.DMA((3,)),
            pltpu.SemaphoreType.DMA((2,)),
        ],
        compiler_params=pltpu.CompilerParams(
            use_tc_tiling_on_sc=False, needs_layout_passes=False),
    )(idx, table)


def kernel(edge_index, embedding_weight):
    idx = edge_index.reshape(-1).astype(jnp.int32)
    out5 = _gather(idx, embedding_weight)
    return jnp.transpose(out5, (0, 2, 4, 1, 3)).reshape(2, _NUM_EDGES, _EMBED_DIM)
